# baseline (speedup 1.0000x reference)
import math

import jax
import jax.numpy as jnp
from jax import lax
from jax.experimental import pallas as pl
from jax.experimental.pallas import tpu as pltpu
from jax.experimental.pallas import tpu_sc as plsc

_N_NODES = 100000
_N_EDGES = 6400000
_P = 1.852
_NC = 2
_NS = 16
_NW = _NC * _NS
_PER_W = _N_EDGES // _NW
_CHUNK = 800
_N_CHUNKS = _PER_W // _CHUNK
_GROUPS = _CHUNK // 16

_C = (1.4426592196, -0.72077582817, 0.48478361101, -0.38679567777,
      0.25271571639)
_SQRT2 = 1.4142135623730951
_LN2 = math.log(2.0)


def _spow_p(x):
    xb = plsc.bitcast(x, jnp.int32)
    xi = lax.bitwise_and(xb, 0x7FFFFFFF)
    e = lax.shift_right_logical(xi, 23) - 127
    mi = lax.bitwise_or(lax.bitwise_and(xi, 0x007FFFFF), 0x3F800000)
    m = plsc.bitcast(mi, jnp.float32)
    big = m >= _SQRT2
    m = jnp.where(big, m * 0.5, m)
    ef = e.astype(jnp.float32) + jnp.where(big, 1.0, 0.0).astype(jnp.float32)
    t = m - 1.0
    poly = jnp.float32(_C[4])
    for c in (_C[3], _C[2], _C[1], _C[0]):
        poly = poly * t + jnp.float32(c)
    log2x = ef + t * poly
    p = jnp.exp(log2x * jnp.float32(_P * _LN2))
    pb = plsc.bitcast(p, jnp.int32)
    sgn = lax.bitwise_and(xb, jnp.int32(-2147483648))
    return plsc.bitcast(lax.bitwise_or(pb, sgn), jnp.float32)


def _edge_body(out_hbm, fr_hbm, to_hbm, virt_hbm, lc_hbm, ny_hbm, ey_hbm,
               o1_hbm, o2_hbm, flg_hbm,
               ny_v, bufs0, bufs1, fl_v, isem0, isem1, osem0, osem1):
    c = lax.axis_index("c")
    s = lax.axis_index("s")
    wid = s * _NC + c
    base_w = wid * _PER_W
    pltpu.sync_copy(ny_hbm, ny_v)

    def in_pairs(base, bufs):
        fr_v, to_v, out_v, ey_v, virt_v, lc_v, _, _ = bufs
        return ((fr_hbm.at[pl.ds(base, _CHUNK)], fr_v),
                (to_hbm.at[pl.ds(base, _CHUNK)], to_v),
                (out_hbm.at[pl.ds(base, _CHUNK)], out_v),
                (ey_hbm.at[pl.ds(base, _CHUNK)], ey_v),
                (virt_hbm.at[pl.ds(base, _CHUNK)], virt_v),
                (lc_hbm.at[pl.ds(base, _CHUNK)], lc_v))

    def fire_in(base, bufs, sem):
        for src, dst in in_pairs(base, bufs):
            pltpu.async_copy(src, dst, sem)

    def wait_in(base, bufs, sem):
        for src, dst in in_pairs(base, bufs):
            pltpu.make_async_copy(src, dst, sem).wait()

    def fire_out(base, bufs, sem):
        o1_v, o2_v = bufs[6], bufs[7]
        pltpu.async_copy(o1_v, o1_hbm.at[pl.ds(base, _CHUNK)], sem)
        pltpu.async_copy(o2_v, o2_hbm.at[pl.ds(base, _CHUNK)], sem)

    def wait_out(base, bufs, sem):
        o1_v, o2_v = bufs[6], bufs[7]
        pltpu.make_async_copy(o1_v, o1_hbm.at[pl.ds(base, _CHUNK)], sem).wait()
        pltpu.make_async_copy(o2_v, o2_hbm.at[pl.ds(base, _CHUNK)], sem).wait()

    def compute(bufs, acc):
        fr_v, to_v, out_v, ey_v, virt_v, lc_v, o1_v, o2_v = bufs

        def grp(g, acc):
          for u in range(2):
            i = g * 2 + u
            sl = pl.ds(i * 16, 16)
            fr = fr_v[sl]
            to = to_v[sl]
            ydiff = plsc.load_gather(ny_v, [to]) - plsc.load_gather(ny_v, [fr])
            virt = virt_v[sl]
            lc = jnp.abs(lc_v[sl])
            o = out_v[sl] + 1e-6
            hl = _spow_p(o) * lc
            is_real = virt == 0.0
            virtual = jnp.where(is_real, 1.0, 0.0).astype(jnp.float32)
            ey = ey_v[sl]
            fle = _spow_p(ey) * lc
            err = jnp.abs(ydiff - fle)
            ah = jnp.abs(ydiff)
            disc = ((err > 0.01) & (err > 0.01 * (ah + 0.01))
                    & (ah > 0.001) & is_real)
            o1_v[sl] = hl * virtual
            o2_v[sl] = ydiff * virtual
            acc = jnp.where(disc, 1.0, acc).astype(jnp.float32)
          return acc

        return lax.fori_loop(0, _GROUPS // 2, grp, acc)

    fire_in(base_w, bufs0, isem0)

    def pair_body(jp, acc):
        b0 = base_w + 2 * jp * _CHUNK
        b1 = b0 + _CHUNK
        fire_in(b1, bufs1, isem1)
        wait_in(b0, bufs0, isem0)

        @pl.when(jp != 0)
        def _():
            wait_out(b0, bufs0, osem0)

        acc = compute(bufs0, acc)
        fire_out(b0, bufs0, osem0)

        @pl.when(jp != _N_CHUNKS // 2 - 1)
        def _():
            fire_in(b1 + _CHUNK, bufs0, isem0)

        wait_in(b1, bufs1, isem1)

        @pl.when(jp != 0)
        def _():
            wait_out(b1, bufs1, osem1)

        acc = compute(bufs1, acc)
        fire_out(b1, bufs1, osem1)
        return acc

    acc = lax.fori_loop(0, _N_CHUNKS // 2, pair_body,
                        jnp.zeros((16,), jnp.float32))
    wait_out(base_w, bufs0, osem0)
    wait_out(base_w, bufs1, osem1)
    fl_v[0, :] = acc
    pltpu.sync_copy(fl_v, flg_hbm.at[pl.ds(wid, 1), :])


def _edge_kernel(output, fr_col, to_col, virt_col, lc_col, node_y, edge_y):
    f32 = jnp.float32
    buf = (
        pltpu.VMEM((_CHUNK,), jnp.int32),
        pltpu.VMEM((_CHUNK,), jnp.int32),
        pltpu.VMEM((_CHUNK,), f32),
        pltpu.VMEM((_CHUNK,), f32),
        pltpu.VMEM((_CHUNK,), f32),
        pltpu.VMEM((_CHUNK,), f32),
        pltpu.VMEM((_CHUNK,), f32),
        pltpu.VMEM((_CHUNK,), f32),
    )
    return pl.kernel(
        _edge_body,
        out_type=(
            jax.ShapeDtypeStruct((_N_EDGES,), f32),
            jax.ShapeDtypeStruct((_N_EDGES,), f32),
            jax.ShapeDtypeStruct((_NW, 16), f32),
        ),
        mesh=plsc.VectorSubcoreMesh(core_axis_name="c", subcore_axis_name="s"),
        compiler_params=pltpu.CompilerParams(needs_layout_passes=False),
        scratch_types=(
            pltpu.VMEM((_N_NODES,), f32),
            buf,
            buf,
            pltpu.VMEM((1, 16), f32),
            pltpu.SemaphoreType.DMA,
            pltpu.SemaphoreType.DMA,
            pltpu.SemaphoreType.DMA,
            pltpu.SemaphoreType.DMA,
        ),
    )(output, fr_col, to_col, virt_col, lc_col, node_y, edge_y)


_ROWS = 50000
_FILL_ROWS = 1000


def _nan_body(flg_ref, o1_in, o2_in, o1_out, o2_out, nan_v, sem):
    bad = jnp.any(flg_ref[...] != 0.0)

    @pl.when(bad)
    def _():
        nan_v[...] = jnp.full((_FILL_ROWS, 128), jnp.nan, jnp.float32)

        def fill(i, carry):
            cp1 = pltpu.make_async_copy(
                nan_v, o1_out.at[pl.ds(i * _FILL_ROWS, _FILL_ROWS), :], sem)
            cp1.start()
            cp1.wait()
            cp2 = pltpu.make_async_copy(
                nan_v, o2_out.at[pl.ds(i * _FILL_ROWS, _FILL_ROWS), :], sem)
            cp2.start()
            cp2.wait()
            return carry

        lax.fori_loop(0, _ROWS // _FILL_ROWS, fill, 0)


def _nan_kernel(flags, o1, o2):
    f32 = jnp.float32
    return pl.pallas_call(
        _nan_body,
        out_shape=(
            jax.ShapeDtypeStruct((_ROWS, 128), f32),
            jax.ShapeDtypeStruct((_ROWS, 128), f32),
        ),
        in_specs=[
            pl.BlockSpec(memory_space=pltpu.VMEM),
            pl.BlockSpec(memory_space=pl.ANY),
            pl.BlockSpec(memory_space=pl.ANY),
        ],
        out_specs=[
            pl.BlockSpec(memory_space=pl.ANY),
            pl.BlockSpec(memory_space=pl.ANY),
        ],
        input_output_aliases={1: 0, 2: 1},
        scratch_shapes=[
            pltpu.VMEM((_FILL_ROWS, 128), f32),
            pltpu.SemaphoreType.DMA,
        ],
    )(flags, o1, o2)


def kernel(output, edge_index, edge_attr, node_y, edge_y):
    fr_col = edge_index[0]
    to_col = edge_index[1]
    virt_col = edge_attr[:, 0]
    lc_col = edge_attr[:, 1]
    o1, o2, flags = _edge_kernel(output, fr_col, to_col, virt_col, lc_col,
                                 node_y, edge_y)
    o1f, o2f = _nan_kernel(flags.reshape(4, 128),
                           o1.reshape(_ROWS, 128), o2.reshape(_ROWS, 128))
    return o1f.reshape(-1), o2f.reshape(-1)


# speedup vs baseline: 1.0651x; 1.0651x over previous
import math

import jax
import jax.numpy as jnp
from jax import lax
from jax.experimental import pallas as pl
from jax.experimental.pallas import tpu as pltpu
from jax.experimental.pallas import tpu_sc as plsc

_N_NODES = 100000
_N_EDGES = 6400000
_P = 1.852
_NC = 2
_NS = 16
_NW = _NC * _NS
_PER_W = _N_EDGES // _NW
_CHUNK = 800
_N_CHUNKS = _PER_W // _CHUNK
_GROUPS = _CHUNK // 16

_C = (1.4426592196, -0.72077582817, 0.48478361101, -0.38679567777,
      0.25271571639)
_SQRT2 = 1.4142135623730951
_LN2 = math.log(2.0)


def _spow_p(x):
    xb = plsc.bitcast(x, jnp.int32)
    xi = lax.bitwise_and(xb, 0x7FFFFFFF)
    e = lax.shift_right_logical(xi, 23) - 127
    mi = lax.bitwise_or(lax.bitwise_and(xi, 0x007FFFFF), 0x3F800000)
    m = plsc.bitcast(mi, jnp.float32)
    big = m >= _SQRT2
    m = jnp.where(big, m * 0.5, m)
    ef = e.astype(jnp.float32) + jnp.where(big, 1.0, 0.0).astype(jnp.float32)
    t = m - 1.0
    poly = jnp.float32(_C[4])
    for c in (_C[3], _C[2], _C[1], _C[0]):
        poly = poly * t + jnp.float32(c)
    log2x = ef + t * poly
    p = jnp.exp(log2x * jnp.float32(_P * _LN2))
    pb = plsc.bitcast(p, jnp.int32)
    sgn = lax.bitwise_and(xb, jnp.int32(-2147483648))
    return plsc.bitcast(lax.bitwise_or(pb, sgn), jnp.float32)


def _edge_body(out_hbm, ei_hbm, virt_hbm, lc_hbm, ny_hbm, ey_hbm,
               o1_hbm, o2_hbm, flg_hbm,
               ny_v, bufs0, bufs1, fl_v, isem0, isem1, osem0, osem1):
    c = lax.axis_index("c")
    s = lax.axis_index("s")
    wid = s * _NC + c
    base_w = wid * _PER_W
    pltpu.sync_copy(ny_hbm, ny_v)

    def in_pairs(base, bufs):
        fr_v, to_v, out_v, ey_v, virt_v, lc_v, _, _ = bufs
        return ((ei_hbm.at[pl.ds(base, _CHUNK)], fr_v),
                (ei_hbm.at[pl.ds(_N_EDGES + base, _CHUNK)], to_v),
                (out_hbm.at[pl.ds(base, _CHUNK)], out_v),
                (ey_hbm.at[pl.ds(base, _CHUNK)], ey_v),
                (virt_hbm.at[pl.ds(base, _CHUNK)], virt_v),
                (lc_hbm.at[pl.ds(base, _CHUNK)], lc_v))

    def fire_in(base, bufs, sem):
        for src, dst in in_pairs(base, bufs):
            pltpu.async_copy(src, dst, sem)

    def wait_in(base, bufs, sem):
        for src, dst in in_pairs(base, bufs):
            pltpu.make_async_copy(src, dst, sem).wait()

    def fire_out(base, bufs, sem):
        o1_v, o2_v = bufs[6], bufs[7]
        pltpu.async_copy(o1_v, o1_hbm.at[pl.ds(base, _CHUNK)], sem)
        pltpu.async_copy(o2_v, o2_hbm.at[pl.ds(base, _CHUNK)], sem)

    def wait_out(base, bufs, sem):
        o1_v, o2_v = bufs[6], bufs[7]
        pltpu.make_async_copy(o1_v, o1_hbm.at[pl.ds(base, _CHUNK)], sem).wait()
        pltpu.make_async_copy(o2_v, o2_hbm.at[pl.ds(base, _CHUNK)], sem).wait()

    def compute(bufs, acc):
        fr_v, to_v, out_v, ey_v, virt_v, lc_v, o1_v, o2_v = bufs

        def grp(g, acc):
          for u in range(2):
            i = g * 2 + u
            sl = pl.ds(i * 16, 16)
            fr = fr_v[sl]
            to = to_v[sl]
            ydiff = plsc.load_gather(ny_v, [to]) - plsc.load_gather(ny_v, [fr])
            virt = virt_v[sl]
            lc = jnp.abs(lc_v[sl])
            o = out_v[sl] + 1e-6
            hl = _spow_p(o) * lc
            is_real = virt == 0.0
            virtual = jnp.where(is_real, 1.0, 0.0).astype(jnp.float32)
            ey = ey_v[sl]
            fle = _spow_p(ey) * lc
            err = jnp.abs(ydiff - fle)
            ah = jnp.abs(ydiff)
            disc = ((err > 0.01) & (err > 0.01 * (ah + 0.01))
                    & (ah > 0.001) & is_real)
            o1_v[sl] = hl * virtual
            o2_v[sl] = ydiff * virtual
            acc = jnp.where(disc, 1.0, acc).astype(jnp.float32)
          return acc

        return lax.fori_loop(0, _GROUPS // 2, grp, acc)

    fire_in(base_w, bufs0, isem0)

    def pair_body(jp, acc):
        b0 = base_w + 2 * jp * _CHUNK
        b1 = b0 + _CHUNK
        fire_in(b1, bufs1, isem1)
        wait_in(b0, bufs0, isem0)

        @pl.when(jp != 0)
        def _():
            wait_out(b0, bufs0, osem0)

        acc = compute(bufs0, acc)
        fire_out(b0, bufs0, osem0)

        @pl.when(jp != _N_CHUNKS // 2 - 1)
        def _():
            fire_in(b1 + _CHUNK, bufs0, isem0)

        wait_in(b1, bufs1, isem1)

        @pl.when(jp != 0)
        def _():
            wait_out(b1, bufs1, osem1)

        acc = compute(bufs1, acc)
        fire_out(b1, bufs1, osem1)
        return acc

    acc = lax.fori_loop(0, _N_CHUNKS // 2, pair_body,
                        jnp.zeros((16,), jnp.float32))
    wait_out(base_w, bufs0, osem0)
    wait_out(base_w, bufs1, osem1)
    fl_v[0, :] = acc
    pltpu.sync_copy(fl_v, flg_hbm.at[pl.ds(wid, 1), :])


def _edge_kernel(output, ei_flat, virt_col, lc_col, node_y, edge_y):
    f32 = jnp.float32
    buf = (
        pltpu.VMEM((_CHUNK,), jnp.int32),
        pltpu.VMEM((_CHUNK,), jnp.int32),
        pltpu.VMEM((_CHUNK,), f32),
        pltpu.VMEM((_CHUNK,), f32),
        pltpu.VMEM((_CHUNK,), f32),
        pltpu.VMEM((_CHUNK,), f32),
        pltpu.VMEM((_CHUNK,), f32),
        pltpu.VMEM((_CHUNK,), f32),
    )
    return pl.kernel(
        _edge_body,
        out_type=(
            jax.ShapeDtypeStruct((_N_EDGES,), f32),
            jax.ShapeDtypeStruct((_N_EDGES,), f32),
            jax.ShapeDtypeStruct((_NW, 16), f32),
        ),
        mesh=plsc.VectorSubcoreMesh(core_axis_name="c", subcore_axis_name="s"),
        compiler_params=pltpu.CompilerParams(needs_layout_passes=False),
        scratch_types=(
            pltpu.VMEM((_N_NODES,), f32),
            buf,
            buf,
            pltpu.VMEM((1, 16), f32),
            pltpu.SemaphoreType.DMA,
            pltpu.SemaphoreType.DMA,
            pltpu.SemaphoreType.DMA,
            pltpu.SemaphoreType.DMA,
        ),
    )(output, ei_flat, virt_col, lc_col, node_y, edge_y)


_ROWS = 50000
_FILL_ROWS = 1000


def _nan_body(flg_ref, o1_in, o2_in, o1_out, o2_out, nan_v, sem):
    bad = jnp.any(flg_ref[...] != 0.0)

    @pl.when(bad)
    def _():
        nan_v[...] = jnp.full((_FILL_ROWS, 128), jnp.nan, jnp.float32)

        def fill(i, carry):
            cp1 = pltpu.make_async_copy(
                nan_v, o1_out.at[pl.ds(i * _FILL_ROWS, _FILL_ROWS), :], sem)
            cp1.start()
            cp1.wait()
            cp2 = pltpu.make_async_copy(
                nan_v, o2_out.at[pl.ds(i * _FILL_ROWS, _FILL_ROWS), :], sem)
            cp2.start()
            cp2.wait()
            return carry

        lax.fori_loop(0, _ROWS // _FILL_ROWS, fill, 0)


def _nan_kernel(flags, o1, o2):
    f32 = jnp.float32
    return pl.pallas_call(
        _nan_body,
        out_shape=(
            jax.ShapeDtypeStruct((_ROWS, 128), f32),
            jax.ShapeDtypeStruct((_ROWS, 128), f32),
        ),
        in_specs=[
            pl.BlockSpec(memory_space=pltpu.VMEM),
            pl.BlockSpec(memory_space=pl.ANY),
            pl.BlockSpec(memory_space=pl.ANY),
        ],
        out_specs=[
            pl.BlockSpec(memory_space=pl.ANY),
            pl.BlockSpec(memory_space=pl.ANY),
        ],
        input_output_aliases={1: 0, 2: 1},
        scratch_shapes=[
            pltpu.VMEM((_FILL_ROWS, 128), f32),
            pltpu.SemaphoreType.DMA,
        ],
    )(flags, o1, o2)


def kernel(output, edge_index, edge_attr, node_y, edge_y):
    ei_flat = edge_index.reshape(-1)
    virt_col = edge_attr[:, 0]
    lc_col = edge_attr[:, 1]
    o1, o2, flags = _edge_kernel(output, ei_flat, virt_col, lc_col,
                                 node_y, edge_y)
    o1f, o2f = _nan_kernel(flags.reshape(4, 128),
                           o1.reshape(_ROWS, 128), o2.reshape(_ROWS, 128))
    return o1f.reshape(-1), o2f.reshape(-1)
